# baseline (speedup 1.0000x reference)
import jax
import jax.numpy as jnp
from jax import lax
from jax.experimental import pallas as pl

N_TOKENS = 1000
ROWS = 1024


def _onehot_block(x_ref, o_ref):
    xcol = x_ref[0]
    iota = lax.broadcasted_iota(jnp.int32, (ROWS, N_TOKENS), 1)
    o_ref[...] = (iota == xcol).astype(o_ref.dtype)


def kernel(x):
    B, T = x.shape
    n = B * T
    nb = n // ROWS
    x3 = x.reshape(nb, ROWS, 1)
    out = pl.pallas_call(
        _onehot_block,
        grid=(nb,),
        in_specs=[pl.BlockSpec((1, ROWS, 1), lambda i: (i, 0, 0))],
        out_specs=pl.BlockSpec((ROWS, N_TOKENS), lambda i: (i, 0)),
        out_shape=jax.ShapeDtypeStruct((n, N_TOKENS), x.dtype),
    )(x3)
    return out.reshape(B, T, N_TOKENS)


# speedup vs baseline: 1.5035x; 1.5035x over previous
import jax
import jax.numpy as jnp
from jax import lax
from jax.experimental import pallas as pl
from jax.experimental.pallas import tpu as pltpu

N_TOKENS = 1000
RC = 64
D = 4


def _body(x_ref, o_ref, *scratch):
    bufs = scratch[:D]
    sems = scratch[D:]
    i = pl.program_id(0)
    nb = pl.num_programs(0)
    xb = x_ref[0]
    iota = lax.broadcasted_iota(jnp.int32, (RC, xb.shape[1], N_TOKENS), 2)
    blk = (iota == xb).astype(o_ref.dtype)

    for b in range(D):
        @pl.when(i % D == b)
        def _():
            @pl.when(i >= D)
            def _():
                pltpu.make_async_copy(
                    bufs[b], o_ref.at[pl.ds(0, RC)], sems[b]).wait()
            bufs[b][...] = blk
            pltpu.make_async_copy(
                bufs[b], o_ref.at[pl.ds(i * RC, RC)], sems[b]).start()

    @pl.when(i == nb - 1)
    def _():
        for b in range(D):
            pltpu.make_async_copy(
                bufs[b], o_ref.at[pl.ds(0, RC)], sems[b]).wait()


def kernel(x):
    B, T = x.shape
    nb = B // RC
    x4 = x.reshape(nb, RC, T, 1)
    out = pl.pallas_call(
        _body,
        grid=(nb,),
        in_specs=[pl.BlockSpec((1, RC, T, 1), lambda i: (i, 0, 0, 0))],
        out_specs=pl.BlockSpec(memory_space=pltpu.MemorySpace.HBM),
        out_shape=jax.ShapeDtypeStruct((B, T, N_TOKENS), x.dtype),
        scratch_shapes=(
            [pltpu.VMEM((RC, T, N_TOKENS), jnp.int32) for _ in range(D)]
            + [pltpu.SemaphoreType.DMA for _ in range(D)]
        ),
    )(x4)
    return out


# speedup vs baseline: 7.1831x; 4.7775x over previous
import jax
import jax.numpy as jnp
from jax import lax
from jax.experimental import pallas as pl

N_TOKENS = 1000


def _onehot_plane(x_ref, o_ref):
    xb = x_ref[...]
    iota = lax.broadcasted_iota(
        jnp.int32, (1, N_TOKENS, x_ref.shape[2]), 1)
    o_ref[...] = (iota == xb).astype(o_ref.dtype)


def kernel(x):
    B, T = x.shape
    xt = x.T.reshape(T, 1, B)
    out_t = pl.pallas_call(
        _onehot_plane,
        grid=(T,),
        in_specs=[pl.BlockSpec((1, 1, B), lambda j: (j, 0, 0))],
        out_specs=pl.BlockSpec((1, N_TOKENS, B), lambda j: (j, 0, 0)),
        out_shape=jax.ShapeDtypeStruct((T, N_TOKENS, B), x.dtype),
    )(xt)
    return jnp.transpose(out_t, (2, 0, 1))
